# Initial kernel scaffold; baseline (speedup 1.0000x reference)
import jax
import jax.numpy as jnp
from jax.experimental import pallas as pl


def kernel(x):
    raise NotImplementedError("write your pallas kernel here")


# speedup vs baseline: 7.2884x; 7.2884x over previous
import numpy as np
import jax
import jax.numpy as jnp
from jax.experimental import pallas as pl

_M32 = 0xFFFFFFFF
_ROTS = ((13, 15, 26, 6), (17, 29, 16, 24))

_B, _NOBJ, _NCLS = 16, 100, 80
_SPAN_ND = 100


def _np_threefry2x32(k0, k1, c0, c1):
    ks = (k0 & _M32, k1 & _M32, (k0 ^ k1 ^ 0x1BD11BDA) & _M32)
    x0 = (c0 + ks[0]) & _M32
    x1 = (c1 + ks[1]) & _M32
    for i in range(5):
        for r in _ROTS[i % 2]:
            x0 = (x0 + x1) & _M32
            x1 = (((x1 << r) | (x1 >> (32 - r))) & _M32) ^ x0
        x0 = (x0 + ks[(i + 1) % 3]) & _M32
        x1 = (x1 + ks[(i + 2) % 3] + i + 1) & _M32
    return x0, x1


def _np_split(key, n):
    return [_np_threefry2x32(key[0], key[1], 0, i) for i in range(n)]


_KEY = (0, 42)
_K1, _K2, _K3, _K4 = _np_split(_KEY, 4)
_K1A, _K1B = _np_split(_K1, 2)
_K4A, _K4B = _np_split(_K4, 2)


def _s32(v):
    v &= _M32
    return v - (1 << 32) if v >= (1 << 31) else v


def _tf_bits(key, c):
    k0, k1 = key[0] & _M32, key[1] & _M32
    ks = (k0, k1, k0 ^ k1 ^ 0x1BD11BDA)
    x0 = jnp.full(c.shape, _s32(k0), jnp.int32)
    x1 = c + jnp.int32(_s32(k1))
    for i in range(5):
        for r in _ROTS[i % 2]:
            x0 = x0 + x1
            x1 = (jax.lax.shift_left(x1, jnp.int32(r))
                  | jax.lax.shift_right_logical(x1, jnp.int32(32 - r))) ^ x0
        x0 = x0 + jnp.int32(_s32(ks[(i + 1) % 3]))
        x1 = x1 + jnp.int32(_s32(ks[(i + 2) % 3] + i + 1))
    return x0 ^ x1


def _erfinv(x):
    w = -jnp.log1p(-x * x)
    ws = w - jnp.float32(2.5)
    p = jnp.full(x.shape, 2.81022636e-08, jnp.float32)
    for coef in (3.43273939e-07, -3.5233877e-06, -4.39150654e-06,
                 0.00021858087, -0.00125372503, -0.00417768164,
                 0.246640727, 1.50140941):
        p = jnp.float32(coef) + p * ws
    wl = jnp.sqrt(w) - jnp.float32(3.0)
    q = jnp.full(x.shape, -0.000200214257, jnp.float32)
    for coef in (0.000100950558, 0.00134934322, -0.00367342844,
                 0.00573950773, -0.0076224613, 0.00943887047,
                 1.00167406, 2.83297682):
        q = jnp.float32(coef) + q * wl
    return jnp.where(w < jnp.float32(5.0), p, q) * x


def _normal_from_bits(bits):
    m = jax.lax.shift_right_logical(bits, jnp.int32(9))
    f = m.astype(jnp.float32) * jnp.float32(2.0 ** -23)
    lo = jnp.float32(np.nextafter(np.float32(-1.0), np.float32(0.0)))
    u = f * (jnp.float32(1.0) - lo) + lo
    u = jnp.maximum(lo, u)
    return jnp.float32(1.4142135381698608) * _erfinv(u)


def _mod_small(v, span):
    q = jnp.floor(v.astype(jnp.float32) * jnp.float32(1.0 / span)
                  + jnp.float32(1e-3))
    return v - q.astype(jnp.int32) * jnp.int32(span)


def _mod32(bits, span):
    hi = jax.lax.shift_right_logical(bits, jnp.int32(16))
    lo = bits & jnp.int32(0xFFFF)
    t = _mod_small(hi, span) * jnp.int32(65536 % span) + _mod_small(lo, span)
    return _mod_small(t, span)


def _randint_from_bits(u, v, span):
    mult = ((65536 % span) ** 2) % span
    t = _mod32(u, span) * jnp.int32(mult) + _mod32(v, span)
    return _mod_small(t, span)


def _rng_body(nd_ref, boxes_ref, scores_ref, classes_ref):
    row = jax.lax.broadcasted_iota(jnp.int32, (_B, 4 * _NOBJ), 0)
    col = jax.lax.broadcasted_iota(jnp.int32, (_B, 4 * _NOBJ), 1)
    boxes_ref[...] = _normal_from_bits(_tf_bits(_K2, row * (4 * _NOBJ) + col))

    row = jax.lax.broadcasted_iota(jnp.int32, (_B, _NOBJ), 0)
    col = jax.lax.broadcasted_iota(jnp.int32, (_B, _NOBJ), 1)
    c = row * _NOBJ + col
    scores_ref[...] = _normal_from_bits(_tf_bits(_K3, c))
    classes_ref[...] = _randint_from_bits(_tf_bits(_K4A, c),
                                          _tf_bits(_K4B, c), _NCLS)
    cn = jax.lax.broadcasted_iota(jnp.int32, (_B, 1), 0)
    nd_ref[...] = _randint_from_bits(_tf_bits(_K1A, cn),
                                     _tf_bits(_K1B, cn), _SPAN_ND)


def _run_rng():
    return pl.pallas_call(
        _rng_body,
        out_shape=(
            jax.ShapeDtypeStruct((_B, 1), jnp.int32),
            jax.ShapeDtypeStruct((_B, 4 * _NOBJ), jnp.float32),
            jax.ShapeDtypeStruct((_B, _NOBJ), jnp.float32),
            jax.ShapeDtypeStruct((_B, _NOBJ), jnp.int32),
        ),
    )()


def kernel(x):
    num_det, boxes_flat, det_scores, det_classes = _run_rng()
    return (num_det,
            boxes_flat.reshape(_B, _NOBJ, 4),
            det_scores,
            det_classes)
